# baseline (speedup 1.0000x reference)
import jax
import jax.numpy as jnp
from jax.experimental import pallas as pl
from jax.experimental.pallas import tpu as pltpu

_BN = 2048


def _mlp_body(gate_ref, x_ref, w1_ref, b1_ref, w2_ref, b2_ref, prior_ref,
              logits_ref, probs_ref, h_ref):
    @pl.when(pl.program_id(0) == 0)
    def _():
        h_ref[...] = jax.nn.relu(
            jnp.dot(x_ref[...], w1_ref[...],
                    preferred_element_type=jnp.float32) + b1_ref[...])

    base = jnp.dot(h_ref[...], w2_ref[...],
                   preferred_element_type=jnp.float32) + b2_ref[...]
    pc = jnp.clip(prior_ref[...], 1e-06, 1.0 - 1e-06)
    prior_logit = jnp.log(pc / (1.0 - pc))
    logits = base + gate_ref[0] * prior_logit
    logits_ref[...] = logits
    probs_ref[...] = jax.nn.sigmoid(logits)


def _topk_body(probs_ref, out_ref, *, k):
    p = probs_ref[...]
    b, n = p.shape
    bits = jax.lax.bitcast_convert_type(p, jnp.int32)

    lo0 = jnp.zeros((b, 1), jnp.int32)
    hi0 = jnp.full((b, 1), 0x3F800001, jnp.int32)

    def body(_, lh):
        lo, hi = lh
        mid = lo + (hi - lo) // 2
        cnt = jnp.sum((bits >= mid).astype(jnp.int32), axis=1, keepdims=True)
        pred = cnt >= k
        return jnp.where(pred, mid, lo), jnp.where(pred, hi, mid)

    lo, _ = jax.lax.fori_loop(0, 30, body, (lo0, hi0))

    gt = bits > lo
    eq = bits == lo
    c_gt = jnp.sum(gt.astype(jnp.int32), axis=1, keepdims=True)
    m = k - c_gt

    idx = jax.lax.broadcasted_iota(jnp.int32, (b, n), 1)
    lo2 = jnp.full((b, 1), -1, jnp.int32)
    hi2 = jnp.full((b, 1), n - 1, jnp.int32)

    def body2(_, lh):
        lo_, hi_ = lh
        mid = lo_ + (hi_ - lo_) // 2
        cnt = jnp.sum((eq & (idx <= mid)).astype(jnp.int32),
                      axis=1, keepdims=True)
        pred = cnt >= m
        return jnp.where(pred, lo_, mid), jnp.where(pred, mid, hi_)

    _, hi2 = jax.lax.fori_loop(0, 13, body2, (lo2, hi2))

    mask = gt | (eq & (idx <= hi2))
    out_ref[...] = mask.astype(jnp.float32)


def kernel(emb, emb_mask, attn_prior, W1, b1, W2, b2, prior_gate):
    del emb_mask
    B, _, D = emb.shape
    H = W1.shape[1]
    N = W2.shape[1]
    K = 32

    x = emb[:, 0, :]
    gate = jax.nn.sigmoid(prior_gate).reshape(1)
    b1_2d = b1.reshape(1, H)
    b2_2d = b2.reshape(1, N)

    grid = N // _BN
    logits, probs = pl.pallas_call(
        _mlp_body,
        grid=(grid,),
        in_specs=[
            pl.BlockSpec(memory_space=pltpu.SMEM),
            pl.BlockSpec((B, D), lambda i: (0, 0)),
            pl.BlockSpec((D, H), lambda i: (0, 0)),
            pl.BlockSpec((1, H), lambda i: (0, 0)),
            pl.BlockSpec((H, _BN), lambda i: (0, i)),
            pl.BlockSpec((1, _BN), lambda i: (0, i)),
            pl.BlockSpec((B, _BN), lambda i: (0, i)),
        ],
        out_specs=[
            pl.BlockSpec((B, _BN), lambda i: (0, i)),
            pl.BlockSpec((B, _BN), lambda i: (0, i)),
        ],
        out_shape=[
            jax.ShapeDtypeStruct((B, N), jnp.float32),
            jax.ShapeDtypeStruct((B, N), jnp.float32),
        ],
        scratch_shapes=[pltpu.VMEM((B, H), jnp.float32)],
        compiler_params=pltpu.CompilerParams(
            dimension_semantics=("arbitrary",)),
    )(gate, x, W1, b1_2d, W2, b2_2d, attn_prior)

    samples = pl.pallas_call(
        lambda pr, o: _topk_body(pr, o, k=K),
        in_specs=[pl.BlockSpec((B, N), lambda: (0, 0))],
        out_specs=pl.BlockSpec((B, N), lambda: (0, 0)),
        out_shape=jax.ShapeDtypeStruct((B, N), jnp.float32),
    )(probs)

    gate_detached = jax.nn.sigmoid(jax.lax.stop_gradient(prior_gate))
    return (samples, probs, logits, probs, gate_detached)


# speedup vs baseline: 1.1720x; 1.1720x over previous
import jax
import jax.numpy as jnp
from jax.experimental import pallas as pl
from jax.experimental.pallas import tpu as pltpu

_BN = 1024
_K = 32


def _topk_mask(bits_ref, samples_ref):
    b, n = bits_ref.shape

    lo0 = jnp.zeros((b, 1), jnp.int32)
    hi0 = jnp.full((b, 1), 0x3F800001, jnp.int32)

    def vbody(_, lh):
        lo, hi = lh
        mid = lo + (hi - lo) // 2
        cnt = jnp.sum((bits_ref[...] >= mid).astype(jnp.int32),
                      axis=1, keepdims=True)
        pred = cnt >= _K
        return jnp.where(pred, mid, lo), jnp.where(pred, hi, mid)

    lo, _ = jax.lax.fori_loop(0, 30, vbody, (lo0, hi0))

    bits = bits_ref[...]
    gt = bits > lo
    c_gt = jnp.sum(gt.astype(jnp.int32), axis=1, keepdims=True)
    m = _K - c_gt

    idx = jax.lax.broadcasted_iota(jnp.int32, (b, n), 1)

    def wcond(st):
        cnt, _ = st
        return jnp.any(cnt < m)

    def wbody(st):
        cnt, j = st
        active = cnt < m
        eq = bits_ref[...] == lo
        cand = jnp.where(eq & (idx > j), idx, n)
        jmin = jnp.min(cand, axis=1, keepdims=True)
        j = jnp.where(active, jmin, j)
        return cnt + active.astype(jnp.int32), j

    _, jf = jax.lax.while_loop(
        wcond, wbody,
        (jnp.zeros((b, 1), jnp.int32), jnp.full((b, 1), -1, jnp.int32)))

    eq = bits == lo
    samples_ref[...] = (gt | (eq & (idx <= jf))).astype(jnp.float32)


def _fused_body(gate_ref, x_ref, w1_ref, b1_ref, w2_ref, b2_ref, prior_ref,
                logits_ref, probs_ref, samples_ref, h_ref, bits_ref):
    i = pl.program_id(0)

    @pl.when(i == 0)
    def _():
        h_ref[...] = jax.nn.relu(
            jnp.dot(x_ref[...], w1_ref[...],
                    preferred_element_type=jnp.float32) + b1_ref[...])

    base = jnp.dot(h_ref[...], w2_ref[...],
                   preferred_element_type=jnp.float32) + b2_ref[...]
    pc = jnp.clip(prior_ref[...], 1e-06, 1.0 - 1e-06)
    prior_logit = jnp.log(pc / (1.0 - pc))
    logits = base + gate_ref[0] * prior_logit
    probs = jax.nn.sigmoid(logits)
    logits_ref[...] = logits
    probs_ref[...] = probs
    bits_ref[:, pl.ds(i * _BN, _BN)] = jax.lax.bitcast_convert_type(
        probs, jnp.int32)

    @pl.when(i == pl.num_programs(0) - 1)
    def _():
        _topk_mask(bits_ref, samples_ref)


def kernel(emb, emb_mask, attn_prior, W1, b1, W2, b2, prior_gate):
    del emb_mask
    B, _, D = emb.shape
    H = W1.shape[1]
    N = W2.shape[1]

    x = emb[:, 0, :]
    gate = jax.nn.sigmoid(prior_gate).reshape(1)
    b1_2d = b1.reshape(1, H)
    b2_2d = b2.reshape(1, N)

    grid = N // _BN
    logits, probs, samples = pl.pallas_call(
        _fused_body,
        grid=(grid,),
        in_specs=[
            pl.BlockSpec(memory_space=pltpu.SMEM),
            pl.BlockSpec((B, D), lambda i: (0, 0)),
            pl.BlockSpec((D, H), lambda i: (0, 0)),
            pl.BlockSpec((1, H), lambda i: (0, 0)),
            pl.BlockSpec((H, _BN), lambda i: (0, i)),
            pl.BlockSpec((1, _BN), lambda i: (0, i)),
            pl.BlockSpec((B, _BN), lambda i: (0, i)),
        ],
        out_specs=[
            pl.BlockSpec((B, _BN), lambda i: (0, i)),
            pl.BlockSpec((B, _BN), lambda i: (0, i)),
            pl.BlockSpec((B, N), lambda i: (0, 0)),
        ],
        out_shape=[
            jax.ShapeDtypeStruct((B, N), jnp.float32),
            jax.ShapeDtypeStruct((B, N), jnp.float32),
            jax.ShapeDtypeStruct((B, N), jnp.float32),
        ],
        scratch_shapes=[
            pltpu.VMEM((B, H), jnp.float32),
            pltpu.VMEM((B, N), jnp.int32),
        ],
        compiler_params=pltpu.CompilerParams(
            dimension_semantics=("arbitrary",)),
    )(gate, x, W1, b1_2d, W2, b2_2d, attn_prior)

    gate_detached = jax.nn.sigmoid(jax.lax.stop_gradient(prior_gate))
    return (samples, probs, logits, probs, gate_detached)
